# Initial kernel scaffold; baseline (speedup 1.0000x reference)
import jax
import jax.numpy as jnp
from jax.experimental import pallas as pl


def kernel(boxes, scores, cls_probs):
    raise NotImplementedError("write your pallas kernel here")


# speedup vs baseline: 100.5359x; 100.5359x over previous
import jax
import jax.numpy as jnp
from jax import lax
from jax.experimental import pallas as pl
from jax.experimental.pallas import tpu as pltpu

N = 5000
NP = 5120
R, C = 40, 128
CONF_T = 0.3
IOU_T = 0.25
IMG = 640.0


def _conf_body(scores_ref, cls_ref, out_ref):
    out_ref[...] = scores_ref[...] * jnp.max(cls_ref[...], axis=1, keepdims=True)


def _conf(scores2d, cls2d):
    return pl.pallas_call(
        _conf_body,
        out_shape=jax.ShapeDtypeStruct((NP, 1), jnp.float32),
    )(scores2d, cls2d)


def _nms_body(cx_ref, cy_ref, w_ref, h_ref, conf_ref, keep_ref, alive_ref):
    cx = cx_ref[...] * IMG
    cy = cy_ref[...] * IMG
    w = w_ref[...] * IMG
    h = h_ref[...] * IMG
    x1 = cx - w / 2.0
    y1 = cy - h / 2.0
    x2 = cx + w / 2.0
    y2 = cy + h / 2.0
    area = (x2 - x1) * (y2 - y1)
    idx2 = (lax.broadcasted_iota(jnp.int32, (R, C), 0) * C
            + lax.broadcasted_iota(jnp.int32, (R, C), 1))
    alive_ref[...] = conf_ref[...]
    keep_ref[...] = jnp.zeros((R, C), jnp.float32)

    def argmax():
        a = alive_ref[...]
        m = jnp.max(a)
        gi = jnp.min(jnp.where(a == m, idx2, jnp.int32(NP)))
        return m, gi

    def body(state):
        _, gi = state
        onehot = idx2 == gi
        keep_ref[...] = jnp.where(onehot, 1.0, keep_ref[...])
        px1 = jnp.sum(jnp.where(onehot, x1, 0.0))
        py1 = jnp.sum(jnp.where(onehot, y1, 0.0))
        px2 = jnp.sum(jnp.where(onehot, x2, 0.0))
        py2 = jnp.sum(jnp.where(onehot, y2, 0.0))
        parea = (px2 - px1) * (py2 - py1)
        iw = jnp.maximum(jnp.minimum(x2, px2) - jnp.maximum(x1, px1), 0.0)
        ih = jnp.maximum(jnp.minimum(y2, py2) - jnp.maximum(y1, py1), 0.0)
        inter = iw * ih
        union = area + parea - inter
        iou = inter / (union + 1e-9)
        sup = (iou > IOU_T) | onehot
        alive_ref[...] = jnp.where(sup, -1.0, alive_ref[...])
        return argmax()

    lax.while_loop(lambda s: s[0] > CONF_T, body, argmax())


def _nms(cxg, cyg, wg, hg, confg):
    return pl.pallas_call(
        _nms_body,
        out_shape=jax.ShapeDtypeStruct((R, C), jnp.float32),
        scratch_shapes=[pltpu.VMEM((R, C), jnp.float32)],
    )(cxg, cyg, wg, hg, confg)


def _asm_body(boxes_ref, conf_ref, keep_ref, out_ref):
    xywh_norm = (boxes_ref[...] * IMG) / IMG
    k = keep_ref[...]
    out_ref[...] = jnp.concatenate([xywh_norm * k, conf_ref[...] * k], axis=1)


def _assemble(boxes, conf, keep):
    return pl.pallas_call(
        _asm_body,
        out_shape=jax.ShapeDtypeStruct((N, 5), jnp.float32),
    )(boxes, conf, keep)


def kernel(boxes, scores, cls_probs):
    scores2d = jnp.pad(scores, (0, NP - N)).reshape(NP, 1)
    cls_p = jnp.pad(cls_probs, ((0, NP - N), (0, 0)))
    conf_col = _conf(scores2d, cls_p)
    bp = jnp.pad(boxes, ((0, NP - N), (0, 0)))
    cxg = bp[:, 0].reshape(R, C)
    cyg = bp[:, 1].reshape(R, C)
    wg = bp[:, 2].reshape(R, C)
    hg = bp[:, 3].reshape(R, C)
    confg = conf_col.reshape(R, C)
    keepg = _nms(cxg, cyg, wg, hg, confg)
    keep = keepg.reshape(NP, 1)[:N]
    out = _assemble(boxes, conf_col[:N], keep)
    return out
